# baseline (speedup 1.0000x reference)
import functools

import jax
import jax.numpy as jnp
from jax import lax
from jax.experimental import pallas as pl
from jax.experimental.pallas import tpu as pltpu
from jax.experimental.pallas import tpu_sc as plsc

_THRESHOLD = 0.05
_N = 16 * 3 * 512 * 512
_NC, _NS, _L = 2, 16, 16
_SHIFT = 14
_B = 81920
_SLICE = _N // _NS
_CH = 4096
_NCH = _SLICE // _CH
_VPC = _CH // _L


def _sc_hist_body(gen_ref, tgt_ref, out_ref, buf0, buf1, table, sem0, sem1):
    core = lax.axis_index("c")
    sub = lax.axis_index("s")
    row = core * _NS + sub
    base = sub * _SLICE

    @pl.loop(0, _B // _L, unroll=8)
    def _zero(i):
        table[pl.ds(i * _L, _L)] = jnp.zeros((_L,), jnp.float32)

    ones = jnp.full((_L,), 1.0, jnp.float32)

    def _process(bref):
        @plsc.parallel_loop(0, _VPC, unroll=16)
        def _vec(j):
            x = bref[pl.ds(j * _L, _L)]
            y = x * jnp.float32(0.5) + jnp.float32(0.5)
            m = y > jnp.float32(_THRESHOLD)
            bits = lax.bitcast_convert_type(y, jnp.int32)
            key = jnp.minimum(lax.shift_right_logical(bits, _SHIFT), _B - 1)
            plsc.addupdate_scatter(table, [key], ones, mask=m)

    def _run(src):
        pltpu.async_copy(src.at[pl.ds(base, _CH)], buf0, sem0)

        @pl.loop(0, _NCH, step=2)
        def _chunks(i):
            @pl.when(i + 1 < _NCH)
            def _():
                pltpu.async_copy(
                    src.at[pl.ds(base + (i + 1) * _CH, _CH)], buf1, sem1)
            pltpu.make_async_copy(
                src.at[pl.ds(base, _CH)], buf0, sem0).wait()
            _process(buf0)

            @pl.when(i + 2 < _NCH)
            def _():
                pltpu.async_copy(
                    src.at[pl.ds(base + (i + 2) * _CH, _CH)], buf0, sem0)

            @pl.when(i + 1 < _NCH)
            def _():
                pltpu.make_async_copy(
                    src.at[pl.ds(base, _CH)], buf1, sem1).wait()
                _process(buf1)

    @pl.when(core == 0)
    def _():
        _run(gen_ref)

    @pl.when(core == 1)
    def _():
        _run(tgt_ref)

    pltpu.sync_copy(table, out_ref.at[row])


_sc_hist = functools.partial(
    pl.kernel,
    out_type=jax.ShapeDtypeStruct((_NC * _NS, _B), jnp.float32),
    mesh=plsc.VectorSubcoreMesh(
        core_axis_name="c", subcore_axis_name="s",
        num_cores=_NC, num_subcores=_NS),
    scratch_types=[
        pltpu.VMEM((_CH,), jnp.float32),
        pltpu.VMEM((_CH,), jnp.float32),
        pltpu.VMEM((_B,), jnp.float32),
        pltpu.SemaphoreType.DMA,
        pltpu.SemaphoreType.DMA,
    ],
    compiler_params=pltpu.CompilerParams(needs_layout_passes=False),
)(_sc_hist_body)


_R, _C = 640, 128


def _flat_cumsum(x):
    k = 1
    while k < _C:
        x = x + jnp.concatenate(
            [jnp.zeros((_R, k), jnp.float32), x[:, : _C - k]], axis=1)
        k *= 2
    rows = jnp.broadcast_to(x[:, _C - 1:], (_R, _C))
    s = rows
    k = 1
    while k < _R:
        s = s + jnp.concatenate(
            [jnp.zeros((k, _C), jnp.float32), s[: _R - k, :]], axis=0)
        k *= 2
    return x + (s - rows)


def _tc_finish_body(h_ref, out_ref):
    h = h_ref[...]
    hg = sum(h[r] for r in range(_NS))
    ht = sum(h[_NS + r] for r in range(_NS))

    ng = _flat_cumsum(hg)
    nt = _flat_cumsum(ht)
    lg = jnp.sum(hg)
    lt = jnp.sum(ht)

    d_out = ng / lg - nt / lt
    d_in = (ng - hg) / lg - (nt - ht) / lt

    idx = (lax.broadcasted_iota(jnp.int32, (_R, _C), 0) * _C
           + lax.broadcasted_iota(jnp.int32, (_R, _C), 1))
    vlo = lax.bitcast_convert_type(lax.shift_left(idx, _SHIFT), jnp.float32)
    vhi = lax.bitcast_convert_type(
        lax.shift_left(idx + 1, _SHIFT), jnp.float32)
    w = vhi - vlo

    a = jnp.abs(d_in)
    b = jnp.abs(d_out)
    trap = jnp.float32(0.5) * (a + b)
    tri = (d_in * d_in + d_out * d_out) / jnp.maximum(
        jnp.float32(2.0) * (a + b), jnp.float32(1e-30))
    contrib = w * jnp.where(d_in * d_out < 0, tri, trap)
    loss = jnp.sum(contrib)
    loss = jnp.where((lg == 0) | (lt == 0), jnp.float32(0.0), loss)
    out_ref[...] = loss.reshape(1, 1)


_tc_finish = pl.pallas_call(
    _tc_finish_body,
    out_shape=jax.ShapeDtypeStruct((1, 1), jnp.float32),
)


def kernel(generated_img, target_img):
    g = generated_img.reshape(_N)
    t = target_img.reshape(_N)
    h = _sc_hist(g, t)
    loss = _tc_finish(h.reshape(_NC * _NS, _R, _C))
    return loss[0, 0]


# speedup vs baseline: 1.4143x; 1.4143x over previous
import functools

import jax
import jax.numpy as jnp
from jax import lax
from jax.experimental import pallas as pl
from jax.experimental.pallas import tpu as pltpu
from jax.experimental.pallas import tpu_sc as plsc

_THRESHOLD = 0.05
_N = 16 * 3 * 512 * 512
_NC, _NS, _L = 2, 16, 16
_SHIFT = 14
_B = 81920
_ROWS, _COLS = 24576, 512
_TROWS = _ROWS // _NS
_CHR = 8
_NCH = _TROWS // _CHR
_VPR = _COLS // _L


def _sc_hist_body(gen_ref, tgt_ref, out_ref, buf0, buf1, table, sem0, sem1):
    core = lax.axis_index("c")
    sub = lax.axis_index("s")
    row = core * _NS + sub
    base = sub * _TROWS

    @pl.loop(0, _B // _L, unroll=8)
    def _zero(i):
        table[pl.ds(i * _L, _L)] = jnp.zeros((_L,), jnp.float32)

    ones = jnp.full((_L,), 1.0, jnp.float32)

    def _process(bref):
        for r in range(_CHR):
            @plsc.parallel_loop(0, _VPR, unroll=8)
            def _vec(j):
                x = bref[r, pl.ds(j * _L, _L)]
                y = x * jnp.float32(0.5) + jnp.float32(0.5)
                m = y > jnp.float32(_THRESHOLD)
                bits = lax.bitcast_convert_type(y, jnp.int32)
                key = jnp.minimum(
                    lax.shift_right_logical(bits, _SHIFT), _B - 1)
                plsc.addupdate_scatter(table, [key], ones, mask=m)

    def _run(src):
        pltpu.async_copy(src.at[pl.ds(base, _CHR), :], buf0, sem0)

        @pl.loop(0, _NCH, step=2)
        def _chunks(i):
            @pl.when(i + 1 < _NCH)
            def _():
                pltpu.async_copy(
                    src.at[pl.ds(base + (i + 1) * _CHR, _CHR), :], buf1, sem1)
            pltpu.make_async_copy(
                src.at[pl.ds(base, _CHR), :], buf0, sem0).wait()
            _process(buf0)

            @pl.when(i + 2 < _NCH)
            def _():
                pltpu.async_copy(
                    src.at[pl.ds(base + (i + 2) * _CHR, _CHR), :], buf0, sem0)

            @pl.when(i + 1 < _NCH)
            def _():
                pltpu.make_async_copy(
                    src.at[pl.ds(base, _CHR), :], buf1, sem1).wait()
                _process(buf1)

    @pl.when(core == 0)
    def _():
        _run(gen_ref)

    @pl.when(core == 1)
    def _():
        _run(tgt_ref)

    pltpu.sync_copy(table, out_ref.at[row])


_sc_hist = functools.partial(
    pl.kernel,
    out_type=jax.ShapeDtypeStruct((_NC * _NS, _B), jnp.float32),
    mesh=plsc.VectorSubcoreMesh(
        core_axis_name="c", subcore_axis_name="s",
        num_cores=_NC, num_subcores=_NS),
    scratch_types=[
        pltpu.VMEM((_CHR, _COLS), jnp.float32),
        pltpu.VMEM((_CHR, _COLS), jnp.float32),
        pltpu.VMEM((_B,), jnp.float32),
        pltpu.SemaphoreType.DMA,
        pltpu.SemaphoreType.DMA,
    ],
    compiler_params=pltpu.CompilerParams(needs_layout_passes=False, use_tc_tiling_on_sc=True),
)(_sc_hist_body)


_R, _C = 640, 128


def _flat_cumsum(x):
    k = 1
    while k < _C:
        x = x + jnp.concatenate(
            [jnp.zeros((_R, k), jnp.float32), x[:, : _C - k]], axis=1)
        k *= 2
    rows = jnp.broadcast_to(x[:, _C - 1:], (_R, _C))
    s = rows
    k = 1
    while k < _R:
        s = s + jnp.concatenate(
            [jnp.zeros((k, _C), jnp.float32), s[: _R - k, :]], axis=0)
        k *= 2
    return x + (s - rows)


def _tc_finish_body(h_ref, out_ref):
    h = h_ref[...]
    hg = sum(h[r] for r in range(_NS))
    ht = sum(h[_NS + r] for r in range(_NS))

    ng = _flat_cumsum(hg)
    nt = _flat_cumsum(ht)
    lg = jnp.sum(hg)
    lt = jnp.sum(ht)

    d_out = ng / lg - nt / lt
    d_in = (ng - hg) / lg - (nt - ht) / lt

    idx = (lax.broadcasted_iota(jnp.int32, (_R, _C), 0) * _C
           + lax.broadcasted_iota(jnp.int32, (_R, _C), 1))
    vlo = lax.bitcast_convert_type(lax.shift_left(idx, _SHIFT), jnp.float32)
    vhi = lax.bitcast_convert_type(
        lax.shift_left(idx + 1, _SHIFT), jnp.float32)
    w = vhi - vlo

    a = jnp.abs(d_in)
    b = jnp.abs(d_out)
    trap = jnp.float32(0.5) * (a + b)
    tri = (d_in * d_in + d_out * d_out) / jnp.maximum(
        jnp.float32(2.0) * (a + b), jnp.float32(1e-30))
    contrib = w * jnp.where(d_in * d_out < 0, tri, trap)
    loss = jnp.sum(contrib)
    loss = jnp.where((lg == 0) | (lt == 0), jnp.float32(0.0), loss)
    out_ref[...] = loss.reshape(1, 1)


_tc_finish = pl.pallas_call(
    _tc_finish_body,
    out_shape=jax.ShapeDtypeStruct((1, 1), jnp.float32),
)


def kernel(generated_img, target_img):
    g = generated_img.reshape(_ROWS, _COLS)
    t = target_img.reshape(_ROWS, _COLS)
    h = _sc_hist(g, t)
    loss = _tc_finish(h.reshape(_NC * _NS, _R, _C))
    return loss[0, 0]


# speedup vs baseline: 1.5875x; 1.1224x over previous
import functools

import jax
import jax.numpy as jnp
from jax import lax
from jax.experimental import pallas as pl
from jax.experimental.pallas import tpu as pltpu
from jax.experimental.pallas import tpu_sc as plsc

_THRESHOLD = 0.05
_N = 16 * 3 * 512 * 512
_NC, _NS, _L = 2, 16, 16
_SHIFT = 14
_B = 81920
_ROWS, _COLS = 24576, 512
_TROWS = _ROWS // _NS
_CHR = 16
_NCH = _TROWS // _CHR
_VPR = _COLS // _L


def _sc_hist_body(gen_ref, tgt_ref, out_ref, buf0, buf1, table, sem0, sem1):
    core = lax.axis_index("c")
    sub = lax.axis_index("s")
    row = core * _NS + sub
    base = sub * _TROWS

    @pl.loop(0, _B // _L, unroll=8)
    def _zero(i):
        table[pl.ds(i * _L, _L)] = jnp.zeros((_L,), jnp.float32)

    ones = jnp.full((_L,), 1.0, jnp.float32)

    def _process(bref):
        for r in range(_CHR):
            @plsc.parallel_loop(0, _VPR, unroll=8)
            def _vec(j):
                x = bref[r, pl.ds(j * _L, _L)]
                y = x * jnp.float32(0.5) + jnp.float32(0.5)
                m = y > jnp.float32(_THRESHOLD)
                bits = lax.bitcast_convert_type(y, jnp.int32)
                key = jnp.minimum(
                    lax.shift_right_logical(bits, _SHIFT), _B - 1)
                plsc.addupdate_scatter(table, [key], ones, mask=m)

    def _run(src):
        pltpu.async_copy(src.at[pl.ds(base, _CHR), :], buf0, sem0)

        @pl.loop(0, _NCH, step=2)
        def _chunks(i):
            @pl.when(i + 1 < _NCH)
            def _():
                pltpu.async_copy(
                    src.at[pl.ds(base + (i + 1) * _CHR, _CHR), :], buf1, sem1)
            pltpu.make_async_copy(
                src.at[pl.ds(base, _CHR), :], buf0, sem0).wait()
            _process(buf0)

            @pl.when(i + 2 < _NCH)
            def _():
                pltpu.async_copy(
                    src.at[pl.ds(base + (i + 2) * _CHR, _CHR), :], buf0, sem0)

            @pl.when(i + 1 < _NCH)
            def _():
                pltpu.make_async_copy(
                    src.at[pl.ds(base, _CHR), :], buf1, sem1).wait()
                _process(buf1)

    @pl.when(core == 0)
    def _():
        _run(gen_ref)

    @pl.when(core == 1)
    def _():
        _run(tgt_ref)

    pltpu.sync_copy(table, out_ref.at[row])


_sc_hist = functools.partial(
    pl.kernel,
    out_type=jax.ShapeDtypeStruct((_NC * _NS, _B), jnp.float32),
    mesh=plsc.VectorSubcoreMesh(
        core_axis_name="c", subcore_axis_name="s",
        num_cores=_NC, num_subcores=_NS),
    scratch_types=[
        pltpu.VMEM((_CHR, _COLS), jnp.float32),
        pltpu.VMEM((_CHR, _COLS), jnp.float32),
        pltpu.VMEM((_B,), jnp.float32),
        pltpu.SemaphoreType.DMA,
        pltpu.SemaphoreType.DMA,
    ],
    compiler_params=pltpu.CompilerParams(needs_layout_passes=False, use_tc_tiling_on_sc=True),
)(_sc_hist_body)


_R, _C = 640, 128


def _flat_cumsum(x):
    k = 1
    while k < _C:
        x = x + jnp.concatenate(
            [jnp.zeros((_R, k), jnp.float32), x[:, : _C - k]], axis=1)
        k *= 2
    rows = jnp.broadcast_to(x[:, _C - 1:], (_R, _C))
    s = rows
    k = 1
    while k < _R:
        s = s + jnp.concatenate(
            [jnp.zeros((k, _C), jnp.float32), s[: _R - k, :]], axis=0)
        k *= 2
    return x + (s - rows)


def _tc_finish_body(h_ref, out_ref):
    h = h_ref[...]
    hg = sum(h[r] for r in range(_NS))
    ht = sum(h[_NS + r] for r in range(_NS))

    ng = _flat_cumsum(hg)
    nt = _flat_cumsum(ht)
    lg = jnp.sum(hg)
    lt = jnp.sum(ht)

    d_out = ng / lg - nt / lt
    d_in = (ng - hg) / lg - (nt - ht) / lt

    idx = (lax.broadcasted_iota(jnp.int32, (_R, _C), 0) * _C
           + lax.broadcasted_iota(jnp.int32, (_R, _C), 1))
    vlo = lax.bitcast_convert_type(lax.shift_left(idx, _SHIFT), jnp.float32)
    vhi = lax.bitcast_convert_type(
        lax.shift_left(idx + 1, _SHIFT), jnp.float32)
    w = vhi - vlo

    a = jnp.abs(d_in)
    b = jnp.abs(d_out)
    trap = jnp.float32(0.5) * (a + b)
    tri = (d_in * d_in + d_out * d_out) / jnp.maximum(
        jnp.float32(2.0) * (a + b), jnp.float32(1e-30))
    contrib = w * jnp.where(d_in * d_out < 0, tri, trap)
    loss = jnp.sum(contrib)
    loss = jnp.where((lg == 0) | (lt == 0), jnp.float32(0.0), loss)
    out_ref[...] = loss.reshape(1, 1)


_tc_finish = pl.pallas_call(
    _tc_finish_body,
    out_shape=jax.ShapeDtypeStruct((1, 1), jnp.float32),
)


def kernel(generated_img, target_img):
    g = generated_img.reshape(_ROWS, _COLS)
    t = target_img.reshape(_ROWS, _COLS)
    h = _sc_hist(g, t)
    loss = _tc_finish(h.reshape(_NC * _NS, _R, _C))
    return loss[0, 0]
